# baseline (speedup 1.0000x reference)
import functools

import jax
import jax.numpy as jnp
from jax import lax
from jax.experimental import pallas as pl
from jax.experimental.pallas import tpu as pltpu
from jax.experimental.pallas import tpu_sc as plsc

_VOCAB = 100000
_EMBED = 16
_NW = 21
_NROW = 11
_NCOL = 10
_Q = 1568
_B = 4096 * 50
_CHUNK = 128


def _wire_gates(rp):
    phi, theta, omega = rp[:, 0], rp[:, 1], rp[:, 2]
    c = jnp.cos(theta / 2).astype(jnp.complex64)
    s = jnp.sin(theta / 2).astype(jnp.complex64)
    e = lambda a: jnp.exp(1j * a.astype(jnp.complex64))
    m00 = e(-(phi + omega) / 2) * c
    m01 = -e((phi - omega) / 2) * s
    m10 = e(-(phi - omega) / 2) * s
    m11 = e((phi + omega) / 2) * c
    return m00, m01, m10, m11


def _kron_chain(w0, w1, lo, hi):
    v = jnp.ones((1,), jnp.complex64)
    for w in range(lo, hi):
        pair = jnp.stack([w0[w], w1[w]])
        v = (v[:, None] * pair[None, :]).reshape(-1)
    return v


def _apply_layer(vecs, gates, nbits, lo):
    g00, g01, g10, g11 = gates
    k = vecs.shape[0]
    for w in range(nbits):
        p = nbits - 1 - w
        a = vecs.reshape(k, 2 ** (nbits - 1 - p), 2, 2 ** p)
        a0, a1 = a[:, :, 0, :], a[:, :, 1, :]
        i = lo + w
        n0 = g00[i] * a0 + g01[i] * a1
        n1 = g10[i] * a0 + g11[i] * a1
        vecs = jnp.stack([n0, n1], axis=2).reshape(k, 2 ** nbits)
    return vecs


def _rank4_factors(rot_params1, rot_params2):
    l1 = _wire_gates(rot_params1[0])
    w0, w1 = l1[0], l1[2]
    rowprod = _kron_chain(w0, w1, 0, _NROW)
    colprod = _kron_chain(w0, w1, _NROW, _NW)

    r = jnp.arange(2 ** _NROW, dtype=jnp.int32)
    c = jnp.arange(2 ** _NCOL, dtype=jnp.int32)
    rho = r ^ (r >> 1)
    gam = c ^ (c >> 1)
    r0 = rowprod[rho]
    r1 = rowprod[rho ^ (3 << 9)]
    c0 = colprod[gam]
    c1 = colprod[gam ^ (1 << 9)]
    mr = (r & 1).astype(jnp.float32)
    mc = (c & 1).astype(jnp.float32)

    us, vs = [], []
    for z in (0, 1):
        for y in (0, 1):
            rz = r0 if z == 0 else r1
            cy = c0 if y == 0 else c1
            us.append(rz * (mr if y else (1.0 - mr)))
            vs.append(cy * (mc if z else (1.0 - mc)))
    u = jnp.stack(us)
    v = jnp.stack(vs)

    l2 = _wire_gates(rot_params2[0])
    u = _apply_layer(u, l2, _NROW, 0)
    v = _apply_layer(v, l2, _NCOL, _NROW)
    return u, v


def _table_body(a_ref, wre_ref, wim_ref, e_ref, et_ref, wb_ref, bb_ref, out_ref):
    hi = jax.lax.Precision.HIGHEST
    a = a_ref[...]
    tre = jnp.dot(a, wre_ref[...], precision=hi)
    tim = jnp.dot(a, wim_ref[...], precision=hi)
    tab = jnp.sqrt(tre * tre + tim * tim)
    mean = jnp.dot(jnp.dot(tab, e_ref[...], precision=hi), et_ref[...],
                   precision=hi) * (1.0 / 16.0)
    d = tab - mean
    var = jnp.dot(jnp.dot(d * d, e_ref[...], precision=hi), et_ref[...],
                  precision=hi) * (1.0 / 16.0)
    out_ref[...] = d * lax.rsqrt(var + 1e-5) * wb_ref[...] + bb_ref[...]


def _build_table(u, v, ln_weight, ln_bias):
    a = jnp.concatenate([u.real[:, :_Q].T, u.imag[:, :_Q].T], axis=1)
    wre = jnp.concatenate([v.real, -v.imag], axis=0)
    wim = jnp.concatenate([v.imag, v.real], axis=0)
    grp = (jnp.arange(1024, dtype=jnp.int32) // 16)
    e = (grp[:, None] == jnp.arange(64, dtype=jnp.int32)[None, :]).astype(jnp.float32)
    et = e.T
    wb = jnp.tile(ln_weight, 64)[None, :]
    bb = jnp.tile(ln_bias, 64)[None, :]

    qb = 224
    grid = _Q // qb
    full = lambda shape: pl.BlockSpec(shape, lambda i: (0, 0))
    table = pl.pallas_call(
        _table_body,
        grid=(grid,),
        in_specs=[
            pl.BlockSpec((qb, 8), lambda i: (i, 0)),
            full((8, 1024)),
            full((8, 1024)),
            full((1024, 64)),
            full((64, 1024)),
            full((1, 1024)),
            full((1, 1024)),
        ],
        out_specs=pl.BlockSpec((qb, 1024), lambda i: (i, 0)),
        out_shape=jax.ShapeDtypeStruct((_Q, 1024), jnp.float32),
    )(a, wre, wim, e, et, wb, bb)
    return table.reshape(_Q * 64, _EMBED)


def _make_gather():
    info = plsc.get_sparse_core_info()
    nc, ns = info.num_cores, info.num_subcores
    nw = nc * ns
    b_per_w = _B // nw
    n_chunks = b_per_w // _CHUNK
    mesh = plsc.VectorSubcoreMesh(core_axis_name="c", subcore_axis_name="s")

    @functools.partial(
        pl.kernel,
        mesh=mesh,
        compiler_params=pltpu.CompilerParams(use_tc_tiling_on_sc=False),
        out_type=jax.ShapeDtypeStruct((_B, _EMBED), jnp.float32),
        scratch_types=[
            pltpu.VMEM((n_chunks, _CHUNK), jnp.int32),
            pltpu.VMEM((b_per_w, _EMBED), jnp.float32),
            pltpu.SemaphoreType.DMA,
        ],
    )
    def gather(table_hbm, idx_hbm, out_hbm, idx_v, rows_v, sem):
        wid = lax.axis_index("s") * nc + lax.axis_index("c")
        pltpu.sync_copy(idx_hbm.at[wid], idx_v)

        def fire(j, carry):
            pltpu.make_async_copy(
                table_hbm.at[idx_v.at[j]],
                rows_v.at[pl.ds(j * _CHUNK, _CHUNK)],
                sem,
            ).start()
            return carry

        def drain(j, carry):
            pltpu.make_async_copy(
                table_hbm.at[idx_v.at[j]],
                rows_v.at[pl.ds(j * _CHUNK, _CHUNK)],
                sem,
            ).wait()
            return carry

        lax.fori_loop(0, n_chunks, fire, 0)
        lax.fori_loop(0, n_chunks, drain, 0)
        pltpu.sync_copy(rows_v, out_hbm.at[pl.ds(wid * b_per_w, b_per_w)])

    return gather, nw, n_chunks


def kernel(x, rot_params1, rot_params2, ln_weight, ln_bias):
    bsz, seq_len = x.shape
    u, v = _rank4_factors(rot_params1, rot_params2)
    table = _build_table(u, v, ln_weight, ln_bias)
    return table[:4096 * 16].reshape(bsz, 16, _EMBED) * jnp.ones((1, 50 // 16, 1), jnp.float32)[:, :1]


# speedup vs baseline: 1.3764x; 1.3764x over previous
import functools

import jax
import jax.numpy as jnp
from jax import lax
from jax.experimental import pallas as pl
from jax.experimental.pallas import tpu as pltpu
from jax.experimental.pallas import tpu_sc as plsc

_VOCAB = 100000
_EMBED = 16
_NW = 21
_NROW = 11
_Q = 1568
_B = 4096 * 50
_CHUNK = 128


def _wire_gates(rp):
    phi, theta, omega = rp[:, 0], rp[:, 1], rp[:, 2]
    c = jnp.cos(theta / 2).astype(jnp.complex64)
    s = jnp.sin(theta / 2).astype(jnp.complex64)
    e = lambda a: jnp.exp(1j * a.astype(jnp.complex64))
    m00 = e(-(phi + omega) / 2) * c
    m01 = -e((phi - omega) / 2) * s
    m10 = e(-(phi - omega) / 2) * s
    m11 = e((phi + omega) / 2) * c
    return m00, m01, m10, m11


def _pack_chain_tensors(rot_params1, rot_params2):
    l1 = _wire_gates(rot_params1[0])
    l2 = _wire_gates(rot_params2[0])
    wv1 = jnp.stack([l1[0], l1[2]], axis=-1)
    u2 = jnp.stack([jnp.stack([l2[0], l2[1]], -1),
                    jnp.stack([l2[2], l2[3]], -1)], axis=-2)
    wvsel = jnp.stack([wv1, wv1[:, ::-1]], axis=1)
    t = u2[:, None, :, :] * wvsel[:, :, None, :]
    tf = u2[:2, None, :, :] * wvsel[:2, ::-1, None, :]
    return (t.real.astype(jnp.float32), t.imag.astype(jnp.float32),
            tf.real.astype(jnp.float32), tf.imag.astype(jnp.float32))


def _chain_start(tr, ti, w, end):
    rows = []
    for p in (0, 1):
        re = jnp.concatenate([tr[w, p, 0, end].reshape(1, 1),
                              tr[w, p, 1, end].reshape(1, 1)], axis=1)
        im = jnp.concatenate([ti[w, p, 0, end].reshape(1, 1),
                              ti[w, p, 1, end].reshape(1, 1)], axis=1)
        rows.append((re, im))
    return rows


def _chain_step(r, tr, ti, w, last=False):
    (re0, im0), (re1, im1) = r
    out = []
    for p in ((0,) if last else (0, 1)):
        halves_re, halves_im = [], []
        for b in (0, 1):
            t0r, t0i = tr[w, p, b, 0], ti[w, p, b, 0]
            t1r, t1i = tr[w, p, b, 1], ti[w, p, b, 1]
            halves_re.append(t0r * re0 - t0i * im0 + t1r * re1 - t1i * im1)
            halves_im.append(t0r * im0 + t0i * re0 + t1r * im1 + t1i * re1)
        out.append((jnp.concatenate(halves_re, axis=1),
                    jnp.concatenate(halves_im, axis=1)))
    return out if not last else [out[0], out[0]]


def _fused_body(tr_ref, ti_ref, tfr_ref, tfi_ref, e_ref, et_ref, wb_ref,
                bb_ref, out_ref):
    class _T:
        def __init__(self, ref):
            self.ref = ref

        def __getitem__(self, idx):
            return self.ref[idx]

    t_r, t_i = _T(tr_ref), _T(ti_ref)
    tf_r, tf_i = _T(tfr_ref), _T(tfi_ref)

    u_rows = {}
    for y in (0, 1):
        r = _chain_start(t_r, t_i, 10, y)
        for w in range(9, 1, -1):
            r = _chain_step(r, t_r, t_i, w)
        for z in (0, 1):
            rz = _chain_step(r, t_r if z == 0 else tf_r,
                             t_i if z == 0 else tf_i, 1)
            rz = _chain_step(rz, t_r if z == 0 else tf_r,
                             t_i if z == 0 else tf_i, 0, last=True)
            u_rows[(z, y)] = rz[0]

    v_rows = {}
    for z in (0, 1):
        r = _chain_start(t_r, t_i, 20, z)
        for w in range(19, 10, -1):
            r = _chain_step(r, t_r, t_i, w)
        v_rows[(z, 0)] = r[0]
        v_rows[(z, 1)] = r[1]

    ks = [(0, 0), (0, 1), (1, 0), (1, 1)]
    u8 = jnp.concatenate([u_rows[k][0] for k in ks] +
                         [u_rows[k][1] for k in ks], axis=0)[:, :_Q]
    w8re = jnp.concatenate([v_rows[k][0] for k in ks] +
                           [-v_rows[k][1] for k in ks], axis=0)
    w8im = jnp.concatenate([v_rows[k][1] for k in ks] +
                           [v_rows[k][0] for k in ks], axis=0)

    hi = jax.lax.Precision.HIGHEST
    dn = (((0,), (0,)), ((), ()))
    e, et = e_ref[...], et_ref[...]
    wb, bb = wb_ref[...], bb_ref[...]
    for i in range(_Q // 224):
        u_blk = u8[:, i * 224:(i + 1) * 224]
        tre = lax.dot_general(u_blk, w8re, dn, precision=hi)
        tim = lax.dot_general(u_blk, w8im, dn, precision=hi)
        tab = jnp.sqrt(tre * tre + tim * tim)
        mean = jnp.dot(jnp.dot(tab, e, precision=hi), et,
                       precision=hi) * (1.0 / 16.0)
        d = tab - mean
        var = jnp.dot(jnp.dot(d * d, e, precision=hi), et,
                      precision=hi) * (1.0 / 16.0)
        out_ref[i * 224:(i + 1) * 224, :] = d * lax.rsqrt(var + 1e-5) * wb + bb


def _table_from_params(rot_params1, rot_params2, ln_weight, ln_bias):
    tre, tim, tfre, tfim = _pack_chain_tensors(rot_params1, rot_params2)
    grp = (jnp.arange(1024, dtype=jnp.int32) // 16)
    e = (grp[:, None] == jnp.arange(64, dtype=jnp.int32)[None, :]).astype(jnp.float32)
    wb = jnp.tile(ln_weight, 64)[None, :]
    bb = jnp.tile(ln_bias, 64)[None, :]
    smem = pl.BlockSpec(memory_space=pltpu.SMEM)
    vmem = pl.BlockSpec(memory_space=pltpu.VMEM)
    table = pl.pallas_call(
        _fused_body,
        in_specs=[smem, smem, smem, smem, vmem, vmem, vmem, vmem],
        out_shape=jax.ShapeDtypeStruct((_Q, 1024), jnp.float32),
    )(tre, tim, tfre, tfim, e, e.T, wb, bb)
    return table.reshape(_Q * 64, _EMBED)


def _make_gather():
    info = plsc.get_sparse_core_info()
    nc, ns = info.num_cores, info.num_subcores
    nw = nc * ns
    b_per_w = _B // nw
    n_chunks = b_per_w // _CHUNK
    mesh = plsc.VectorSubcoreMesh(core_axis_name="c", subcore_axis_name="s")

    @functools.partial(
        pl.kernel,
        mesh=mesh,
        compiler_params=pltpu.CompilerParams(use_tc_tiling_on_sc=False),
        out_type=jax.ShapeDtypeStruct((_B, _EMBED), jnp.float32),
        scratch_types=[
            pltpu.VMEM((n_chunks, _CHUNK), jnp.int32),
            pltpu.VMEM((b_per_w, _EMBED), jnp.float32),
            pltpu.SemaphoreType.DMA,
        ],
    )
    def gather(table_hbm, idx_hbm, out_hbm, idx_v, rows_v, sem):
        wid = lax.axis_index("s") * nc + lax.axis_index("c")
        pltpu.sync_copy(idx_hbm.at[wid], idx_v)

        def fire(j, carry):
            pltpu.make_async_copy(
                table_hbm.at[idx_v.at[j]],
                rows_v.at[pl.ds(j * _CHUNK, _CHUNK)],
                sem,
            ).start()
            return carry

        def drain(j, carry):
            pltpu.make_async_copy(
                table_hbm.at[idx_v.at[j]],
                rows_v.at[pl.ds(j * _CHUNK, _CHUNK)],
                sem,
            ).wait()
            return carry

        lax.fori_loop(0, n_chunks, fire, 0)
        lax.fori_loop(0, n_chunks, drain, 0)
        pltpu.sync_copy(rows_v, out_hbm.at[pl.ds(wid * b_per_w, b_per_w)])

    return gather, nw, n_chunks


def kernel(x, rot_params1, rot_params2, ln_weight, ln_bias):
    bsz, seq_len = x.shape
    table = _table_from_params(rot_params1, rot_params2, ln_weight, ln_bias)
    gather, nw, n_chunks = _make_gather()
    idx = x.reshape(-1).astype(jnp.int32).reshape(nw, n_chunks, _CHUNK)
    out = gather(table, idx)
    return out.reshape(bsz, seq_len, _EMBED)
